# baseline (speedup 1.0000x reference)
import jax
import jax.numpy as jnp
from jax.experimental import pallas as pl
from jax.experimental.pallas import tpu as pltpu

_MARGIN = 0.1
_ROWS, _COLS = 4096, 1024
_BLK = 512
_NSTEPS = _ROWS // _BLK


def _body(c_ref, m_ref, loss_ref, tgt_ref, nontgt_ref, acc_ref):
    i = pl.program_id(0)
    c = c_ref[...]
    m = m_ref[...]
    a = jnp.min(jnp.where(m, c, jnp.inf))
    b = jnp.max(jnp.where(m, -jnp.inf, c))

    @pl.when(i == 0)
    def _():
        acc_ref[0, 0] = a
        acc_ref[0, 1] = b

    @pl.when(i > 0)
    def _():
        acc_ref[0, 0] = jnp.minimum(acc_ref[0, 0], a)
        acc_ref[0, 1] = jnp.maximum(acc_ref[0, 1], b)

    @pl.when(i == _NSTEPS - 1)
    def _():
        A = acc_ref[0, 0]
        B = acc_ref[0, 1]
        lt = jnp.float32(1.0) - A
        ln = jnp.where(B == -jnp.inf, -jnp.inf,
                       jnp.maximum(B - jnp.float32(_MARGIN), jnp.float32(0.0)))
        tgt_ref[0, 0] = lt
        nontgt_ref[0, 0] = ln
        loss_ref[0, 0] = (lt + ln) * jnp.float32(0.5)


def kernel(cos_pred, mask_gt):
    out_shape = [jax.ShapeDtypeStruct((1, 1), jnp.float32)] * 3
    smem_out = pl.BlockSpec(memory_space=pltpu.SMEM)
    loss, tgt, nontgt = pl.pallas_call(
        _body,
        grid=(_NSTEPS,),
        in_specs=[
            pl.BlockSpec((_BLK, _COLS), lambda i: (i, 0)),
            pl.BlockSpec((_BLK, _COLS), lambda i: (i, 0)),
        ],
        out_specs=[smem_out, smem_out, smem_out],
        out_shape=out_shape,
        scratch_shapes=[pltpu.SMEM((1, 2), jnp.float32)],
    )(cos_pred, mask_gt)
    return (loss[0, 0], tgt[0, 0], nontgt[0, 0])
